# baseline (speedup 1.0000x reference)
import functools

import jax
import jax.numpy as jnp
from jax import lax
from jax.experimental import pallas as pl
from jax.experimental.pallas import tpu as pltpu
from jax.experimental.pallas import tpu_sc as plsc

F32 = jnp.float32
I32 = jnp.int32

N_NODES = 10000
N_EDGES = 160000
E2 = 2 * N_EDGES
NG = 256
HD = 128
ED = 64
NB = 1000
EB = 1000
NNB = N_NODES // NB
NEB = E2 // EB

NW = 32
E2P = 327680
PER_W = E2P // NW
CHG = 128
NCHG = PER_W // CHG
PER_T = E2 // 16
CH = 80
NCHS = PER_T // CH
N_PAD = 10240
HALF = N_PAD // 2
ACC_ROWS = HALF + 128
ZR = ACC_ROWS // 16
OUTR = HALF // 16


def _bcast(shape):
    return pl.BlockSpec(shape, lambda *_: tuple(0 for _ in shape))


def _dot(a, b):
    return jnp.dot(a, b, preferred_element_type=F32)


def _onehot(idx, num):
    return (idx[:, None] ==
            lax.broadcasted_iota(I32, (idx.shape[0], num), 1)).astype(F32)


def _embed_h_body(h_ref, hjc_ref, tbl_ref, out_ref):
    hv = h_ref[0, 0]
    jv = hjc_ref[0, 0]
    h1 = _dot(_onehot(hv, 300), tbl_ref[...])
    h2 = _dot(_onehot(jv, 300), tbl_ref[...])
    out_ref[...] = jnp.concatenate([h1, h2], axis=1)


def _embed_h(h3, hjc3, tbl):
    return pl.pallas_call(
        _embed_h_body,
        grid=(NNB,),
        in_specs=[
            pl.BlockSpec((1, 1, NB), lambda i: (i, 0, 0)),
            pl.BlockSpec((1, 1, NB), lambda i: (i, 0, 0)),
            _bcast((300, 64)),
        ],
        out_specs=pl.BlockSpec((NB, HD), lambda i: (i, 0)),
        out_shape=jax.ShapeDtypeStruct((N_NODES, HD), F32),
    )(h3, hjc3, tbl)


def _embed_e_body(e_ref, tbl_ref, out_ref):
    ev = e_ref[0, 0]
    out_ref[...] = _dot(_onehot(ev, 5), tbl_ref[...])


def _embed_e(e3, tbl):
    return pl.pallas_call(
        _embed_e_body,
        grid=(NEB,),
        in_specs=[
            pl.BlockSpec((1, 1, EB), lambda i: (i, 0, 0)),
            _bcast((5, ED)),
        ],
        out_specs=pl.BlockSpec((EB, ED), lambda i: (i, 0)),
        out_shape=jax.ShapeDtypeStruct((E2, ED), F32),
    )(e3, tbl)


def _embed_g_body(g_ref, w_ref, b_ref, out_ref):
    out_ref[...] = _dot(g_ref[...], w_ref[...]) + b_ref[...]


def _embed_g(g, w, b):
    return pl.pallas_call(
        _embed_g_body,
        in_specs=[_bcast((NG, 64)), _bcast((64, HD)), _bcast((1, HD))],
        out_specs=_bcast((NG, HD)),
        out_shape=jax.ShapeDtypeStruct((NG, HD), F32),
    )(g, w, b.reshape(1, HD))


def _proj_body(h_ref, b_ref, g_ref, whs_ref, whd_ref, wg_ref, eb1_ref,
               asrc_ref, bdst_ref):
    h = h_ref[...]
    oh = _onehot(b_ref[0, 0], NG)
    gproj = _dot(g_ref[...], wg_ref[...])
    asrc_ref[...] = _dot(h, whs_ref[...])
    bdst_ref[...] = _dot(h, whd_ref[...]) + _dot(oh, gproj) + eb1_ref[...]


def _proj(h_embd, batch3, g_embd, whs, whd, wg, eb1):
    return pl.pallas_call(
        _proj_body,
        grid=(NNB,),
        in_specs=[
            pl.BlockSpec((NB, HD), lambda i: (i, 0)),
            pl.BlockSpec((1, 1, NB), lambda i: (i, 0, 0)),
            _bcast((NG, HD)),
            _bcast((HD, HD)),
            _bcast((HD, HD)),
            _bcast((HD, HD)),
            _bcast((1, HD)),
        ],
        out_specs=[
            pl.BlockSpec((NB, HD), lambda i: (i, 0)),
            pl.BlockSpec((NB, HD), lambda i: (i, 0)),
        ],
        out_shape=[jax.ShapeDtypeStruct((N_NODES, HD), F32)] * 2,
    )(h_embd, batch3, g_embd, whs, whd, wg, eb1.reshape(1, HD))


def _edge_body(a_ref, b_ref, e_ref, we_ref, w2_ref, eb2_ref, out_ref):
    e = e_ref[...]
    hid = jax.nn.relu(a_ref[...] + b_ref[...] + _dot(e, we_ref[...]))
    out_ref[...] = e + _dot(hid, w2_ref[...]) + eb2_ref[...]


def _edge_mlp(A, B, e_feat, we, w2, eb2):
    return pl.pallas_call(
        _edge_body,
        grid=(NEB,),
        in_specs=[
            pl.BlockSpec((EB, HD), lambda i: (i, 0)),
            pl.BlockSpec((EB, HD), lambda i: (i, 0)),
            pl.BlockSpec((EB, ED), lambda i: (i, 0)),
            _bcast((ED, HD)),
            _bcast((HD, ED)),
            _bcast((1, ED)),
        ],
        out_specs=pl.BlockSpec((EB, ED), lambda i: (i, 0)),
        out_shape=jax.ShapeDtypeStruct((E2, ED), F32),
    )(A, B, e_feat, we, w2, eb2.reshape(1, ED))


def _node_body(h_ref, a0_ref, b_ref, g_ref, wh_ref, wa_ref, wg_ref,
               nb1_ref, w2_ref, nb2_ref, out_ref):
    h = h_ref[...]
    agg = a0_ref[...]
    oh = _onehot(b_ref[0, 0], NG)
    gproj = _dot(g_ref[...], wg_ref[...])
    hid = jax.nn.relu(_dot(h, wh_ref[...]) + _dot(agg, wa_ref[...]) +
                      _dot(oh, gproj) + nb1_ref[...])
    out_ref[...] = h + _dot(hid, w2_ref[...]) + nb2_ref[...]


def _node_mlp(h_embd, agg0, batch3, g_embd, wh, wa, wg, nb1, w2, nb2):
    return pl.pallas_call(
        _node_body,
        grid=(NNB,),
        in_specs=[
            pl.BlockSpec((NB, HD), lambda i: (i, 0)),
            pl.BlockSpec((NB, ED), lambda i: (i, 0)),
            pl.BlockSpec((1, 1, NB), lambda i: (i, 0, 0)),
            _bcast((NG, HD)),
            _bcast((HD, HD)),
            _bcast((ED, HD)),
            _bcast((HD, HD)),
            _bcast((1, HD)),
            _bcast((HD, HD)),
            _bcast((1, HD)),
        ],
        out_specs=pl.BlockSpec((NB, HD), lambda i: (i, 0)),
        out_shape=jax.ShapeDtypeStruct((N_NODES, HD), F32),
    )(h_embd, agg0, batch3, g_embd, wh, wa, wg,
      nb1.reshape(1, HD), w2, nb2.reshape(1, HD))


def _graph_body(h_ref, b_ref, g_ref, wg_ref, whm_ref, gb1_ref, w2_ref,
                gb2_ref, out_ref, hsum_ref, cnt_ref):
    i = pl.program_id(0)

    @pl.when(i == 0)
    def _():
        hsum_ref[...] = jnp.zeros_like(hsum_ref)
        cnt_ref[...] = jnp.zeros_like(cnt_ref)

    oh = _onehot(b_ref[0, 0], NG)
    contract0 = (((0,), (0,)), ((), ()))
    hsum_ref[...] += lax.dot_general(oh, h_ref[...], contract0,
                                     preferred_element_type=F32)
    cnt_ref[...] += lax.dot_general(oh, jnp.ones((NB, HD), F32), contract0,
                                    preferred_element_type=F32)

    @pl.when(i == NNB - 1)
    def _():
        g = g_ref[...]
        hmean = hsum_ref[...] / jnp.maximum(cnt_ref[...], 1.0)
        hid = jax.nn.relu(_dot(g, wg_ref[...]) + _dot(hmean, whm_ref[...]) +
                          gb1_ref[...])
        out_ref[...] = g + _dot(hid, w2_ref[...]) + gb2_ref[...]


def _graph_mlp(h_new, batch3, g_embd, wg, whm, gb1, w2, gb2):
    return pl.pallas_call(
        _graph_body,
        grid=(NNB,),
        in_specs=[
            pl.BlockSpec((NB, HD), lambda i: (i, 0)),
            pl.BlockSpec((1, 1, NB), lambda i: (i, 0, 0)),
            _bcast((NG, HD)),
            _bcast((HD, HD)),
            _bcast((HD, HD)),
            _bcast((1, HD)),
            _bcast((HD, HD)),
            _bcast((1, HD)),
        ],
        out_specs=_bcast((NG, HD)),
        out_shape=jax.ShapeDtypeStruct((NG, HD), F32),
        scratch_shapes=[pltpu.VMEM((NG, HD), F32), pltpu.VMEM((NG, HD), F32)],
    )(h_new, batch3, g_embd, wg, whm, gb1.reshape(1, HD), w2,
      gb2.reshape(1, HD))


def _readout_body(g_ref, w1_ref, b1_ref, w2_ref, b2_ref, out_ref):
    hid = jax.nn.relu(_dot(g_ref[...], w1_ref[...]) + b1_ref[...])
    out_ref[...] = _dot(hid, w2_ref[...]) + b2_ref[...]


def _readout(g_embd, w1, b1, w2, b2):
    return pl.pallas_call(
        _readout_body,
        in_specs=[_bcast((NG, HD)), _bcast((HD, HD)), _bcast((1, HD)),
                  _bcast((HD, HD)), _bcast((1, HD))],
        out_specs=_bcast((NG, HD)),
        out_shape=jax.ShapeDtypeStruct((NG, HD), F32),
    )(g_embd, w1, b1.reshape(1, HD), w2, b2.reshape(1, HD))


def _pipe2(n, start_p, wait_p, start_c, wait_c):
    start_p(0, 0)

    def body(k, carry):
        j0 = 2 * k

        @pl.when(k > 0)
        def _():
            wait_c(1)

        start_p(j0 + 1, 1)
        wait_p(0)
        start_c(j0, 0)

        @pl.when(k < n // 2 - 1)
        def _():
            wait_c(0)
            start_p(j0 + 2, 0)

        wait_p(1)
        start_c(j0 + 1, 1)
        return carry

    lax.fori_loop(0, n // 2, body, 0)
    wait_c(0)
    wait_c(1)


@functools.lru_cache(maxsize=None)
def _sc_kernels():
    mesh = plsc.VectorSubcoreMesh(core_axis_name="c", subcore_axis_name="s")

    @functools.partial(
        pl.kernel,
        mesh=mesh,
        out_type=[jax.ShapeDtypeStruct((E2P, HD), F32)] * 2,
        scratch_types=[
            pltpu.VMEM((PER_W,), I32),
            pltpu.VMEM((PER_W,), I32),
            pltpu.VMEM((CHG, HD), F32),
            pltpu.VMEM((CHG, HD), F32),
            pltpu.VMEM((CHG, HD), F32),
            pltpu.VMEM((CHG, HD), F32),
            pltpu.SemaphoreType.DMA,
            pltpu.SemaphoreType.DMA,
            pltpu.SemaphoreType.DMA,
            pltpu.SemaphoreType.DMA,
        ],
    )
    def gather_ab(asrc, bdst, src_i, dst_i, out_a, out_b,
                  idx_s, idx_d, ra0, ra1, rb0, rb1, sp0, sp1, sc0, sc1):
        wid = lax.axis_index("s") * 2 + lax.axis_index("c")
        base = wid * PER_W
        pltpu.sync_copy(src_i.at[pl.ds(base, PER_W)], idx_s)
        pltpu.sync_copy(dst_i.at[pl.ds(base, PER_W)], idx_d)
        ra = (ra0, ra1)
        rb = (rb0, rb1)
        sp = (sp0, sp1)
        sc = (sc0, sc1)

        def body(j, carry):
            off = base + j * CHG
            ia = idx_s.at[pl.ds(j * CHG, CHG)]
            ib = idx_d.at[pl.ds(j * CHG, CHG)]
            ca = pltpu.async_copy(asrc.at[ia], ra[0], sp[0])
            cb = pltpu.async_copy(bdst.at[ib], rb[0], sp[1])
            ca.wait()
            cb.wait()
            wa = pltpu.async_copy(ra[0], out_a.at[pl.ds(off, CHG)], sc[0])
            wb = pltpu.async_copy(rb[0], out_b.at[pl.ds(off, CHG)], sc[1])
            wa.wait()
            wb.wait()
            return carry

        lax.fori_loop(0, NCHG, body, 0)

    @functools.partial(
        pl.kernel,
        mesh=mesh,
        out_type=jax.ShapeDtypeStruct((N_PAD, ED), F32),
        scratch_types=[
            pltpu.VMEM((PER_T,), I32),
            pltpu.VMEM((CH, ED), F32),
            pltpu.VMEM((CH, ED), F32),
            pltpu.VMEM_SHARED((ACC_ROWS, ED), F32),
            pltpu.SemaphoreType.DMA,
            pltpu.SemaphoreType.DMA,
            pltpu.SemaphoreType.DMA,
            pltpu.SemaphoreType.DMA,
        ],
        compiler_params=pltpu.CompilerParams(use_tc_tiling_on_sc=False),
    )
    def scatter_agg(e_new, dst2, zeros_hbm, out,
                    idx_d, r0, r1, acc, sp0, sp1, sc0, sc1):
        cid = lax.axis_index("c")
        sid = lax.axis_index("s")
        lo = cid * HALF
        pltpu.sync_copy(dst2.at[pl.ds(cid * E2 + sid * PER_T, PER_T)], idx_d)

        @pl.when(sid == 0)
        def _():
            pltpu.sync_copy(zeros_hbm, acc)

        plsc.subcore_barrier()
        rr = (r0, r1)
        sp = (sp0, sp1)
        sc = (sc0, sc1)

        def body(j, carry):
            pltpu.sync_copy(e_new.at[pl.ds(sid * PER_T + j * CH, CH)], rr[0])
            pltpu.sync_copy(rr[0], acc.at[idx_d.at[pl.ds(j * CH, CH)]],
                            add=True)
            return carry

        lax.fori_loop(0, NCHS, body, 0)
        plsc.subcore_barrier()
        pltpu.sync_copy(acc.at[pl.ds(sid * OUTR, OUTR)],
                        out.at[pl.ds(lo + sid * OUTR, OUTR)])

    return gather_ab, scatter_agg


def _gather_ab(asrc, bdst, src_i, dst_i):
    return _sc_kernels()[0](asrc, bdst, src_i, dst_i)


def _scatter_agg(e_new, dst_i, zeros_hbm):
    return _sc_kernels()[1](e_new, dst_i, zeros_hbm)


def kernel(h, h_junction_count, e_index, e, g, batch, params):
    h3 = h.astype(I32).reshape(NNB, 1, NB)
    hjc3 = h_junction_count.astype(I32).reshape(NNB, 1, NB)
    batch3 = batch.astype(I32).reshape(NNB, 1, NB)
    src_i = jnp.concatenate([e_index[0], e_index[1]]).astype(I32)
    dst_i = jnp.concatenate([e_index[1], e_index[0]]).astype(I32)
    pad = jnp.zeros((E2P - E2,), I32)
    src_ip = jnp.concatenate([src_i, pad])
    dst_ip = jnp.concatenate([dst_i, pad])
    e3 = jnp.concatenate([e, e]).astype(I32).reshape(NEB, 1, EB)
    zeros_hbm = jnp.zeros((ACC_ROWS, ED), F32)
    dst_c0 = jnp.where(dst_i < HALF, dst_i, HALF)
    dst_c1 = jnp.where(dst_i >= HALF, dst_i - HALF, HALF)
    dst2 = jnp.concatenate([dst_c0, dst_c1])

    p = params
    h_embd = _embed_h(h3, hjc3, p["embd_h"])
    e_feat = _embed_e(e3, p["embd_e"])
    g_embd = _embed_g(g, p["embd_g_W"], p["embd_g_b"])

    for lp in p["layers"]:
        eW1 = lp["eW1"]
        asrc, bdst = _proj(h_embd, batch3, g_embd,
                           eW1[0:128], eW1[128:256], eW1[320:448], lp["eb1"])
        A, B = _gather_ab(asrc, bdst, src_ip, dst_ip)
        e_feat = _edge_mlp(A, B, e_feat, eW1[256:320], lp["eW2"], lp["eb2"])
        agg2 = _scatter_agg(e_feat, dst2, zeros_hbm)
        nW1 = lp["nW1"]
        h_embd = _node_mlp(h_embd, agg2[:N_NODES],
                           batch3, g_embd, nW1[0:128], nW1[128:192],
                           nW1[192:320], lp["nb1"], lp["nW2"], lp["nb2"])
        gW1 = lp["gW1"]
        g_embd = _graph_mlp(h_embd, batch3, g_embd, gW1[0:128], gW1[128:256],
                            lp["gb1"], lp["gW2"], lp["gb2"])

    return _readout(g_embd, p["roW1"], p["rob1"], p["roW2"], p["rob2"])


# speedup vs baseline: 1.1557x; 1.1557x over previous
import functools

import jax
import jax.numpy as jnp
from jax import lax
from jax.experimental import pallas as pl
from jax.experimental.pallas import tpu as pltpu
from jax.experimental.pallas import tpu_sc as plsc

F32 = jnp.float32
I32 = jnp.int32

N_NODES = 10000
N_EDGES = 160000
E2 = 2 * N_EDGES
NG = 256
HD = 128
ED = 64
NB = 1000
EB = 1000
NNB = N_NODES // NB
NEB = E2 // EB

NW = 32
E2P = 327680
PER_W = E2P // NW
CHG = 128
NCHG = PER_W // CHG
PER_T = E2 // 16
CH = 80
NCHS = PER_T // CH
N_PAD = 10240
HALF = N_PAD // 2
ACC_ROWS = HALF + 128
ZR = ACC_ROWS // 16
OUTR = HALF // 16


def _bcast(shape):
    return pl.BlockSpec(shape, lambda *_: tuple(0 for _ in shape))


def _dot(a, b):
    return jnp.dot(a, b, preferred_element_type=F32)


def _onehot(idx, num):
    return (idx[:, None] ==
            lax.broadcasted_iota(I32, (idx.shape[0], num), 1)).astype(F32)


def _embed_h_body(h_ref, hjc_ref, tbl_ref, out_ref):
    hv = h_ref[0, 0]
    jv = hjc_ref[0, 0]
    h1 = _dot(_onehot(hv, 300), tbl_ref[...])
    h2 = _dot(_onehot(jv, 300), tbl_ref[...])
    out_ref[...] = jnp.concatenate([h1, h2], axis=1)


def _embed_h(h3, hjc3, tbl):
    return pl.pallas_call(
        _embed_h_body,
        grid=(NNB,),
        in_specs=[
            pl.BlockSpec((1, 1, NB), lambda i: (i, 0, 0)),
            pl.BlockSpec((1, 1, NB), lambda i: (i, 0, 0)),
            _bcast((300, 64)),
        ],
        out_specs=pl.BlockSpec((NB, HD), lambda i: (i, 0)),
        out_shape=jax.ShapeDtypeStruct((N_NODES, HD), F32),
    )(h3, hjc3, tbl)


def _embed_e_body(e_ref, tbl_ref, out_ref):
    ev = e_ref[0, 0]
    out_ref[...] = _dot(_onehot(ev, 5), tbl_ref[...])


def _embed_e(e3, tbl):
    return pl.pallas_call(
        _embed_e_body,
        grid=(NEB,),
        in_specs=[
            pl.BlockSpec((1, 1, EB), lambda i: (i, 0, 0)),
            _bcast((5, ED)),
        ],
        out_specs=pl.BlockSpec((EB, ED), lambda i: (i, 0)),
        out_shape=jax.ShapeDtypeStruct((E2, ED), F32),
    )(e3, tbl)


def _embed_g_body(g_ref, w_ref, b_ref, out_ref):
    out_ref[...] = _dot(g_ref[...], w_ref[...]) + b_ref[...]


def _embed_g(g, w, b):
    return pl.pallas_call(
        _embed_g_body,
        in_specs=[_bcast((NG, 64)), _bcast((64, HD)), _bcast((1, HD))],
        out_specs=_bcast((NG, HD)),
        out_shape=jax.ShapeDtypeStruct((NG, HD), F32),
    )(g, w, b.reshape(1, HD))


def _proj_body(h_ref, b_ref, g_ref, whs_ref, whd_ref, wg_ref, eb1_ref,
               asrc_ref, bdst_ref):
    h = h_ref[...]
    oh = _onehot(b_ref[0, 0], NG)
    gproj = _dot(g_ref[...], wg_ref[...])
    asrc_ref[...] = _dot(h, whs_ref[...])
    bdst_ref[...] = _dot(h, whd_ref[...]) + _dot(oh, gproj) + eb1_ref[...]


def _proj(h_embd, batch3, g_embd, whs, whd, wg, eb1):
    return pl.pallas_call(
        _proj_body,
        grid=(NNB,),
        in_specs=[
            pl.BlockSpec((NB, HD), lambda i: (i, 0)),
            pl.BlockSpec((1, 1, NB), lambda i: (i, 0, 0)),
            _bcast((NG, HD)),
            _bcast((HD, HD)),
            _bcast((HD, HD)),
            _bcast((HD, HD)),
            _bcast((1, HD)),
        ],
        out_specs=[
            pl.BlockSpec((NB, HD), lambda i: (i, 0)),
            pl.BlockSpec((NB, HD), lambda i: (i, 0)),
        ],
        out_shape=[jax.ShapeDtypeStruct((N_NODES, HD), F32)] * 2,
    )(h_embd, batch3, g_embd, whs, whd, wg, eb1.reshape(1, HD))


def _edge_body(a_ref, b_ref, e_ref, we_ref, w2_ref, eb2_ref, out_ref):
    e = e_ref[...]
    hid = jax.nn.relu(a_ref[...] + b_ref[...] + _dot(e, we_ref[...]))
    out_ref[...] = e + _dot(hid, w2_ref[...]) + eb2_ref[...]


def _edge_mlp(A, B, e_feat, we, w2, eb2):
    return pl.pallas_call(
        _edge_body,
        grid=(NEB,),
        in_specs=[
            pl.BlockSpec((EB, HD), lambda i: (i, 0)),
            pl.BlockSpec((EB, HD), lambda i: (i, 0)),
            pl.BlockSpec((EB, ED), lambda i: (i, 0)),
            _bcast((ED, HD)),
            _bcast((HD, ED)),
            _bcast((1, ED)),
        ],
        out_specs=pl.BlockSpec((EB, ED), lambda i: (i, 0)),
        out_shape=jax.ShapeDtypeStruct((E2, ED), F32),
    )(A, B, e_feat, we, w2, eb2.reshape(1, ED))


def _node_body(h_ref, a0_ref, b_ref, g_ref, wh_ref, wa_ref, wg_ref,
               nb1_ref, w2_ref, nb2_ref, out_ref):
    h = h_ref[...]
    agg = a0_ref[...]
    oh = _onehot(b_ref[0, 0], NG)
    gproj = _dot(g_ref[...], wg_ref[...])
    hid = jax.nn.relu(_dot(h, wh_ref[...]) + _dot(agg, wa_ref[...]) +
                      _dot(oh, gproj) + nb1_ref[...])
    out_ref[...] = h + _dot(hid, w2_ref[...]) + nb2_ref[...]


def _node_mlp(h_embd, agg0, batch3, g_embd, wh, wa, wg, nb1, w2, nb2):
    return pl.pallas_call(
        _node_body,
        grid=(NNB,),
        in_specs=[
            pl.BlockSpec((NB, HD), lambda i: (i, 0)),
            pl.BlockSpec((NB, ED), lambda i: (i, 0)),
            pl.BlockSpec((1, 1, NB), lambda i: (i, 0, 0)),
            _bcast((NG, HD)),
            _bcast((HD, HD)),
            _bcast((ED, HD)),
            _bcast((HD, HD)),
            _bcast((1, HD)),
            _bcast((HD, HD)),
            _bcast((1, HD)),
        ],
        out_specs=pl.BlockSpec((NB, HD), lambda i: (i, 0)),
        out_shape=jax.ShapeDtypeStruct((N_NODES, HD), F32),
    )(h_embd, agg0, batch3, g_embd, wh, wa, wg,
      nb1.reshape(1, HD), w2, nb2.reshape(1, HD))


def _graph_body(h_ref, b_ref, g_ref, wg_ref, whm_ref, gb1_ref, w2_ref,
                gb2_ref, out_ref, hsum_ref, cnt_ref):
    i = pl.program_id(0)

    @pl.when(i == 0)
    def _():
        hsum_ref[...] = jnp.zeros_like(hsum_ref)
        cnt_ref[...] = jnp.zeros_like(cnt_ref)

    oh = _onehot(b_ref[0, 0], NG)
    contract0 = (((0,), (0,)), ((), ()))
    hsum_ref[...] += lax.dot_general(oh, h_ref[...], contract0,
                                     preferred_element_type=F32)
    cnt_ref[...] += lax.dot_general(oh, jnp.ones((NB, HD), F32), contract0,
                                    preferred_element_type=F32)

    @pl.when(i == NNB - 1)
    def _():
        g = g_ref[...]
        hmean = hsum_ref[...] / jnp.maximum(cnt_ref[...], 1.0)
        hid = jax.nn.relu(_dot(g, wg_ref[...]) + _dot(hmean, whm_ref[...]) +
                          gb1_ref[...])
        out_ref[...] = g + _dot(hid, w2_ref[...]) + gb2_ref[...]


def _graph_mlp(h_new, batch3, g_embd, wg, whm, gb1, w2, gb2):
    return pl.pallas_call(
        _graph_body,
        grid=(NNB,),
        in_specs=[
            pl.BlockSpec((NB, HD), lambda i: (i, 0)),
            pl.BlockSpec((1, 1, NB), lambda i: (i, 0, 0)),
            _bcast((NG, HD)),
            _bcast((HD, HD)),
            _bcast((HD, HD)),
            _bcast((1, HD)),
            _bcast((HD, HD)),
            _bcast((1, HD)),
        ],
        out_specs=_bcast((NG, HD)),
        out_shape=jax.ShapeDtypeStruct((NG, HD), F32),
        scratch_shapes=[pltpu.VMEM((NG, HD), F32), pltpu.VMEM((NG, HD), F32)],
    )(h_new, batch3, g_embd, wg, whm, gb1.reshape(1, HD), w2,
      gb2.reshape(1, HD))


def _readout_body(g_ref, w1_ref, b1_ref, w2_ref, b2_ref, out_ref):
    hid = jax.nn.relu(_dot(g_ref[...], w1_ref[...]) + b1_ref[...])
    out_ref[...] = _dot(hid, w2_ref[...]) + b2_ref[...]


def _readout(g_embd, w1, b1, w2, b2):
    return pl.pallas_call(
        _readout_body,
        in_specs=[_bcast((NG, HD)), _bcast((HD, HD)), _bcast((1, HD)),
                  _bcast((HD, HD)), _bcast((1, HD))],
        out_specs=_bcast((NG, HD)),
        out_shape=jax.ShapeDtypeStruct((NG, HD), F32),
    )(g_embd, w1, b1.reshape(1, HD), w2, b2.reshape(1, HD))


def _pipe2(n, start_p, wait_p, start_c, wait_c):
    start_p(0, 0)

    def body(k, carry):
        j0 = 2 * k

        @pl.when(k > 0)
        def _():
            wait_c(1)

        start_p(j0 + 1, 1)
        wait_p(0)
        start_c(j0, 0)

        @pl.when(k < n // 2 - 1)
        def _():
            wait_c(0)
            start_p(j0 + 2, 0)

        wait_p(1)
        start_c(j0 + 1, 1)
        return carry

    lax.fori_loop(0, n // 2, body, 0)
    wait_c(0)
    wait_c(1)


@functools.lru_cache(maxsize=None)
def _sc_kernels():
    mesh = plsc.VectorSubcoreMesh(core_axis_name="c", subcore_axis_name="s")

    @functools.partial(
        pl.kernel,
        mesh=mesh,
        out_type=[jax.ShapeDtypeStruct((E2, HD), F32)] * 2,
        scratch_types=[
            pltpu.VMEM((CH,), I32),
            pltpu.VMEM((CH,), I32),
            pltpu.VMEM((CH, HD), F32),
            pltpu.VMEM((CH, HD), F32),
            pltpu.SemaphoreType.DMA,
        ],
    )
    def gather_ab(asrc, bdst, src_i, dst_i, out_a, out_b,
                  idx_s, idx_d, rows_a, rows_b, sem):
        wid = lax.axis_index("s") * 2 + lax.axis_index("c")
        base = wid * (E2 // NW)

        def body(j, carry):
            off = base + j * CH
            pltpu.sync_copy(src_i.at[pl.ds(off, CH)], idx_s)
            pltpu.sync_copy(dst_i.at[pl.ds(off, CH)], idx_d)
            ca = pltpu.async_copy(asrc.at[idx_s], rows_a, sem)
            cb = pltpu.async_copy(bdst.at[idx_d], rows_b, sem)
            ca.wait()
            cb.wait()
            pltpu.sync_copy(rows_a, out_a.at[pl.ds(off, CH)])
            pltpu.sync_copy(rows_b, out_b.at[pl.ds(off, CH)])
            return carry

        lax.fori_loop(0, (E2 // NW) // CH, body, 0)

    @functools.partial(
        pl.kernel,
        mesh=mesh,
        out_type=jax.ShapeDtypeStruct((N_PAD, ED), F32),
        scratch_types=[
            pltpu.VMEM((CH,), I32),
            pltpu.VMEM((CH, ED), F32),
            pltpu.VMEM_SHARED((ACC_ROWS, ED), F32),
        ],
        compiler_params=pltpu.CompilerParams(use_tc_tiling_on_sc=False),
    )
    def scatter_agg(e_new, dst2, zeros_hbm, out, idx_d, rows, acc):
        cid = lax.axis_index("c")
        sid = lax.axis_index("s")
        lo = cid * HALF

        @pl.when(sid == 0)
        def _():
            pltpu.sync_copy(zeros_hbm, acc)

        plsc.subcore_barrier()

        def body(j, carry):
            off = cid * E2 + sid * PER_T + j * CH
            pltpu.sync_copy(dst2.at[pl.ds(off, CH)], idx_d)
            pltpu.sync_copy(e_new.at[pl.ds(sid * PER_T + j * CH, CH)], rows)
            pltpu.sync_copy(rows, acc.at[idx_d], add=True)
            return carry

        lax.fori_loop(0, NCHS, body, 0)
        plsc.subcore_barrier()
        pltpu.sync_copy(acc.at[pl.ds(sid * OUTR, OUTR)],
                        out.at[pl.ds(lo + sid * OUTR, OUTR)])

    return gather_ab, scatter_agg


def _gather_ab(asrc, bdst, src_i, dst_i):
    return _sc_kernels()[0](asrc, bdst, src_i, dst_i)


def _scatter_agg(e_new, dst_i, zeros_hbm):
    return _sc_kernels()[1](e_new, dst_i, zeros_hbm)


def kernel(h, h_junction_count, e_index, e, g, batch, params):
    h3 = h.astype(I32).reshape(NNB, 1, NB)
    hjc3 = h_junction_count.astype(I32).reshape(NNB, 1, NB)
    batch3 = batch.astype(I32).reshape(NNB, 1, NB)
    src_i = jnp.concatenate([e_index[0], e_index[1]]).astype(I32)
    dst_i = jnp.concatenate([e_index[1], e_index[0]]).astype(I32)
    e3 = jnp.concatenate([e, e]).astype(I32).reshape(NEB, 1, EB)
    zeros_hbm = jnp.zeros((ACC_ROWS, ED), F32)
    dst_c0 = jnp.where(dst_i < HALF, dst_i, HALF)
    dst_c1 = jnp.where(dst_i >= HALF, dst_i - HALF, HALF)
    dst2 = jnp.concatenate([dst_c0, dst_c1])

    p = params
    h_embd = _embed_h(h3, hjc3, p["embd_h"])
    e_feat = _embed_e(e3, p["embd_e"])
    g_embd = _embed_g(g, p["embd_g_W"], p["embd_g_b"])

    for lp in p["layers"]:
        eW1 = lp["eW1"]
        asrc, bdst = _proj(h_embd, batch3, g_embd,
                           eW1[0:128], eW1[128:256], eW1[320:448], lp["eb1"])
        A, B = _gather_ab(asrc, bdst, src_i, dst_i)
        e_feat = _edge_mlp(A, B, e_feat, eW1[256:320], lp["eW2"], lp["eb2"])
        agg2 = _scatter_agg(e_feat, dst2, zeros_hbm)
        nW1 = lp["nW1"]
        h_embd = _node_mlp(h_embd, agg2[:N_NODES],
                           batch3, g_embd, nW1[0:128], nW1[128:192],
                           nW1[192:320], lp["nb1"], lp["nW2"], lp["nb2"])
        gW1 = lp["gW1"]
        g_embd = _graph_mlp(h_embd, batch3, g_embd, gW1[0:128], gW1[128:256],
                            lp["gb1"], lp["gW2"], lp["gb2"])

    return _readout(g_embd, p["roW1"], p["rob1"], p["roW2"], p["rob2"])


# speedup vs baseline: 1.1560x; 1.0003x over previous
import functools

import jax
import jax.numpy as jnp
from jax import lax
from jax.experimental import pallas as pl
from jax.experimental.pallas import tpu as pltpu
from jax.experimental.pallas import tpu_sc as plsc

F32 = jnp.float32
I32 = jnp.int32

N_NODES = 10000
N_EDGES = 160000
E2 = 2 * N_EDGES
NG = 256
HD = 128
ED = 64
NB = 1000
EB = 1000
NNB = N_NODES // NB
NEB = E2 // EB

NW = 32
PER_T = E2 // 16
CH = 80
NCHS = PER_T // CH
N_PAD = 10240
HALF = N_PAD // 2
ACC_ROWS = HALF + 128
OUTR = HALF // 16


def _bcast(shape):
    return pl.BlockSpec(shape, lambda *_: tuple(0 for _ in shape))


def _dot(a, b):
    return jnp.dot(a, b, preferred_element_type=F32)


def _onehot(idx, num):
    return (idx[:, None] ==
            lax.broadcasted_iota(I32, (idx.shape[0], num), 1)).astype(F32)


def _embed_h_body(h_ref, hjc_ref, tbl_ref, out_ref):
    hv = h_ref[0, 0]
    jv = hjc_ref[0, 0]
    h1 = _dot(_onehot(hv, 300), tbl_ref[...])
    h2 = _dot(_onehot(jv, 300), tbl_ref[...])
    out_ref[...] = jnp.concatenate([h1, h2], axis=1)


def _embed_h(h3, hjc3, tbl):
    return pl.pallas_call(
        _embed_h_body,
        grid=(NNB,),
        in_specs=[
            pl.BlockSpec((1, 1, NB), lambda i: (i, 0, 0)),
            pl.BlockSpec((1, 1, NB), lambda i: (i, 0, 0)),
            _bcast((300, 64)),
        ],
        out_specs=pl.BlockSpec((NB, HD), lambda i: (i, 0)),
        out_shape=jax.ShapeDtypeStruct((N_NODES, HD), F32),
    )(h3, hjc3, tbl)


def _embed_e_body(e_ref, tbl_ref, out_ref):
    ev = e_ref[0, 0]
    out_ref[...] = _dot(_onehot(ev, 5), tbl_ref[...])


def _embed_e(e3, tbl):
    return pl.pallas_call(
        _embed_e_body,
        grid=(NEB,),
        in_specs=[
            pl.BlockSpec((1, 1, EB), lambda i: (i, 0, 0)),
            _bcast((5, ED)),
        ],
        out_specs=pl.BlockSpec((EB, ED), lambda i: (i, 0)),
        out_shape=jax.ShapeDtypeStruct((E2, ED), F32),
    )(e3, tbl)


def _embed_g_body(g_ref, w_ref, b_ref, out_ref):
    out_ref[...] = _dot(g_ref[...], w_ref[...]) + b_ref[...]


def _embed_g(g, w, b):
    return pl.pallas_call(
        _embed_g_body,
        in_specs=[_bcast((NG, 64)), _bcast((64, HD)), _bcast((1, HD))],
        out_specs=_bcast((NG, HD)),
        out_shape=jax.ShapeDtypeStruct((NG, HD), F32),
    )(g, w, b.reshape(1, HD))


def _proj_body(h_ref, b_ref, g_ref, whs_ref, whd_ref, wg_ref, eb1_ref,
               asrc_ref, bdst_ref):
    h = h_ref[...]
    oh = _onehot(b_ref[0, 0], NG)
    gproj = _dot(g_ref[...], wg_ref[...])
    asrc_ref[...] = _dot(h, whs_ref[...])
    bdst_ref[...] = _dot(h, whd_ref[...]) + _dot(oh, gproj) + eb1_ref[...]


def _proj(h_embd, batch3, g_embd, whs, whd, wg, eb1):
    return pl.pallas_call(
        _proj_body,
        grid=(NNB,),
        in_specs=[
            pl.BlockSpec((NB, HD), lambda i: (i, 0)),
            pl.BlockSpec((1, 1, NB), lambda i: (i, 0, 0)),
            _bcast((NG, HD)),
            _bcast((HD, HD)),
            _bcast((HD, HD)),
            _bcast((HD, HD)),
            _bcast((1, HD)),
        ],
        out_specs=[
            pl.BlockSpec((NB, HD), lambda i: (i, 0)),
            pl.BlockSpec((NB, HD), lambda i: (i, 0)),
        ],
        out_shape=[jax.ShapeDtypeStruct((N_NODES, HD), F32)] * 2,
    )(h_embd, batch3, g_embd, whs, whd, wg, eb1.reshape(1, HD))


def _edge_body(a_ref, b_ref, e_ref, we_ref, w2_ref, eb2_ref, out_ref):
    e = e_ref[...]
    hid = jax.nn.relu(a_ref[...] + b_ref[...] + _dot(e, we_ref[...]))
    out_ref[...] = e + _dot(hid, w2_ref[...]) + eb2_ref[...]


def _edge_mlp(A, B, e_feat, we, w2, eb2):
    return pl.pallas_call(
        _edge_body,
        grid=(NEB,),
        in_specs=[
            pl.BlockSpec((EB, HD), lambda i: (i, 0)),
            pl.BlockSpec((EB, HD), lambda i: (i, 0)),
            pl.BlockSpec((EB, ED), lambda i: (i, 0)),
            _bcast((ED, HD)),
            _bcast((HD, ED)),
            _bcast((1, ED)),
        ],
        out_specs=pl.BlockSpec((EB, ED), lambda i: (i, 0)),
        out_shape=jax.ShapeDtypeStruct((E2, ED), F32),
    )(A, B, e_feat, we, w2, eb2.reshape(1, ED))


def _node_body(h_ref, a0_ref, b_ref, g_ref, wh_ref, wa_ref, wg_ref,
               nb1_ref, w2_ref, nb2_ref, out_ref):
    h = h_ref[...]
    agg = a0_ref[...]
    oh = _onehot(b_ref[0, 0], NG)
    gproj = _dot(g_ref[...], wg_ref[...])
    hid = jax.nn.relu(_dot(h, wh_ref[...]) + _dot(agg, wa_ref[...]) +
                      _dot(oh, gproj) + nb1_ref[...])
    out_ref[...] = h + _dot(hid, w2_ref[...]) + nb2_ref[...]


def _node_mlp(h_embd, agg0, batch3, g_embd, wh, wa, wg, nb1, w2, nb2):
    return pl.pallas_call(
        _node_body,
        grid=(NNB,),
        in_specs=[
            pl.BlockSpec((NB, HD), lambda i: (i, 0)),
            pl.BlockSpec((NB, ED), lambda i: (i, 0)),
            pl.BlockSpec((1, 1, NB), lambda i: (i, 0, 0)),
            _bcast((NG, HD)),
            _bcast((HD, HD)),
            _bcast((ED, HD)),
            _bcast((HD, HD)),
            _bcast((1, HD)),
            _bcast((HD, HD)),
            _bcast((1, HD)),
        ],
        out_specs=pl.BlockSpec((NB, HD), lambda i: (i, 0)),
        out_shape=jax.ShapeDtypeStruct((N_NODES, HD), F32),
    )(h_embd, agg0, batch3, g_embd, wh, wa, wg,
      nb1.reshape(1, HD), w2, nb2.reshape(1, HD))


def _graph_body(h_ref, b_ref, g_ref, wg_ref, whm_ref, gb1_ref, w2_ref,
                gb2_ref, out_ref, hsum_ref, cnt_ref):
    i = pl.program_id(0)

    @pl.when(i == 0)
    def _():
        hsum_ref[...] = jnp.zeros_like(hsum_ref)
        cnt_ref[...] = jnp.zeros_like(cnt_ref)

    oh = _onehot(b_ref[0, 0], NG)
    contract0 = (((0,), (0,)), ((), ()))
    hsum_ref[...] += lax.dot_general(oh, h_ref[...], contract0,
                                     preferred_element_type=F32)
    cnt_ref[...] += lax.dot_general(oh, jnp.ones((NB, HD), F32), contract0,
                                    preferred_element_type=F32)

    @pl.when(i == NNB - 1)
    def _():
        g = g_ref[...]
        hmean = hsum_ref[...] / jnp.maximum(cnt_ref[...], 1.0)
        hid = jax.nn.relu(_dot(g, wg_ref[...]) + _dot(hmean, whm_ref[...]) +
                          gb1_ref[...])
        out_ref[...] = g + _dot(hid, w2_ref[...]) + gb2_ref[...]


def _graph_mlp(h_new, batch3, g_embd, wg, whm, gb1, w2, gb2):
    return pl.pallas_call(
        _graph_body,
        grid=(NNB,),
        in_specs=[
            pl.BlockSpec((NB, HD), lambda i: (i, 0)),
            pl.BlockSpec((1, 1, NB), lambda i: (i, 0, 0)),
            _bcast((NG, HD)),
            _bcast((HD, HD)),
            _bcast((HD, HD)),
            _bcast((1, HD)),
            _bcast((HD, HD)),
            _bcast((1, HD)),
        ],
        out_specs=_bcast((NG, HD)),
        out_shape=jax.ShapeDtypeStruct((NG, HD), F32),
        scratch_shapes=[pltpu.VMEM((NG, HD), F32), pltpu.VMEM((NG, HD), F32)],
    )(h_new, batch3, g_embd, wg, whm, gb1.reshape(1, HD), w2,
      gb2.reshape(1, HD))


def _readout_body(g_ref, w1_ref, b1_ref, w2_ref, b2_ref, out_ref):
    hid = jax.nn.relu(_dot(g_ref[...], w1_ref[...]) + b1_ref[...])
    out_ref[...] = _dot(hid, w2_ref[...]) + b2_ref[...]


def _readout(g_embd, w1, b1, w2, b2):
    return pl.pallas_call(
        _readout_body,
        in_specs=[_bcast((NG, HD)), _bcast((HD, HD)), _bcast((1, HD)),
                  _bcast((HD, HD)), _bcast((1, HD))],
        out_specs=_bcast((NG, HD)),
        out_shape=jax.ShapeDtypeStruct((NG, HD), F32),
    )(g_embd, w1, b1.reshape(1, HD), w2, b2.reshape(1, HD))


@functools.lru_cache(maxsize=None)
def _sc_kernels():
    mesh = plsc.VectorSubcoreMesh(core_axis_name="c", subcore_axis_name="s")

    @functools.partial(
        pl.kernel,
        mesh=mesh,
        out_type=[jax.ShapeDtypeStruct((E2, HD), F32)] * 2,
        scratch_types=[
            pltpu.VMEM((CH,), I32),
            pltpu.VMEM((CH,), I32),
            pltpu.VMEM((CH, HD), F32),
            pltpu.VMEM((CH, HD), F32),
            pltpu.SemaphoreType.DMA,
        ],
    )
    def gather_ab(asrc, bdst, src_i, dst_i, out_a, out_b,
                  idx_s, idx_d, rows_a, rows_b, sem):
        wid = lax.axis_index("s") * 2 + lax.axis_index("c")
        base = wid * (E2 // NW)

        def body(j, carry):
            off = base + j * CH
            pltpu.sync_copy(src_i.at[pl.ds(off, CH)], idx_s)
            pltpu.sync_copy(dst_i.at[pl.ds(off, CH)], idx_d)
            ca = pltpu.async_copy(asrc.at[idx_s], rows_a, sem)
            cb = pltpu.async_copy(bdst.at[idx_d], rows_b, sem)
            ca.wait()
            cb.wait()
            pltpu.sync_copy(rows_a, out_a.at[pl.ds(off, CH)])
            pltpu.sync_copy(rows_b, out_b.at[pl.ds(off, CH)])
            return carry

        lax.fori_loop(0, (E2 // NW) // CH, body, 0)

    @functools.partial(
        pl.kernel,
        mesh=mesh,
        out_type=jax.ShapeDtypeStruct((N_PAD, ED), F32),
        scratch_types=[
            pltpu.VMEM((CH,), I32),
            pltpu.VMEM((CH, ED), F32),
            pltpu.VMEM_SHARED((ACC_ROWS, ED), F32),
        ],
        compiler_params=pltpu.CompilerParams(use_tc_tiling_on_sc=False),
    )
    def scatter_agg(e_new, dst2, zeros_hbm, out, idx_d, rows, acc):
        cid = lax.axis_index("c")
        sid = lax.axis_index("s")
        lo = cid * HALF

        @pl.when(sid == 0)
        def _():
            pltpu.sync_copy(zeros_hbm, acc)

        plsc.subcore_barrier()

        def body(j, carry):
            off = cid * E2 + sid * PER_T + j * CH
            pltpu.sync_copy(dst2.at[pl.ds(off, CH)], idx_d)
            pltpu.sync_copy(e_new.at[pl.ds(sid * PER_T + j * CH, CH)], rows)
            pltpu.sync_copy(rows, acc.at[idx_d], add=True)
            return carry

        lax.fori_loop(0, NCHS, body, 0)
        plsc.subcore_barrier()
        pltpu.sync_copy(acc.at[pl.ds(sid * OUTR, OUTR)],
                        out.at[pl.ds(lo + sid * OUTR, OUTR)])

    return gather_ab, scatter_agg


def _gather_ab(asrc, bdst, src_i, dst_i):
    return _sc_kernels()[0](asrc, bdst, src_i, dst_i)


def _scatter_agg(e_new, dst_i, zeros_hbm):
    return _sc_kernels()[1](e_new, dst_i, zeros_hbm)


def kernel(h, h_junction_count, e_index, e, g, batch, params):
    h3 = h.astype(I32).reshape(NNB, 1, NB)
    hjc3 = h_junction_count.astype(I32).reshape(NNB, 1, NB)
    batch3 = batch.astype(I32).reshape(NNB, 1, NB)
    src_i = jnp.concatenate([e_index[0], e_index[1]]).astype(I32)
    dst_i = jnp.concatenate([e_index[1], e_index[0]]).astype(I32)
    e3 = jnp.concatenate([e, e]).astype(I32).reshape(NEB, 1, EB)
    zeros_hbm = jnp.zeros((ACC_ROWS, ED), F32)
    dst_c0 = jnp.where(dst_i < HALF, dst_i, HALF)
    dst_c1 = jnp.where(dst_i >= HALF, dst_i - HALF, HALF)
    dst2 = jnp.concatenate([dst_c0, dst_c1])

    p = params
    h_embd = _embed_h(h3, hjc3, p["embd_h"])
    e_feat = _embed_e(e3, p["embd_e"])
    g_embd = _embed_g(g, p["embd_g_W"], p["embd_g_b"])

    for lp in p["layers"]:
        eW1 = lp["eW1"]
        asrc, bdst = _proj(h_embd, batch3, g_embd,
                           eW1[0:128], eW1[128:256], eW1[320:448], lp["eb1"])
        A, B = _gather_ab(asrc, bdst, src_i, dst_i)
        e_feat = _edge_mlp(A, B, e_feat, eW1[256:320], lp["eW2"], lp["eb2"])
        agg2 = _scatter_agg(e_feat, dst2, zeros_hbm)
        nW1 = lp["nW1"]
        h_embd = _node_mlp(h_embd, agg2[:N_NODES],
                           batch3, g_embd, nW1[0:128], nW1[128:192],
                           nW1[192:320], lp["nb1"], lp["nW2"], lp["nb2"])
        gW1 = lp["gW1"]
        g_embd = _graph_mlp(h_embd, batch3, g_embd, gW1[0:128], gW1[128:256],
                            lp["gb1"], lp["gW2"], lp["gb2"])

    return _readout(g_embd, p["roW1"], p["rob1"], p["roW2"], p["rob2"])
